# initial kernel scaffold (unmeasured)
import jax
import jax.numpy as jnp
from jax import lax
from jax.experimental import pallas as pl
from jax.experimental.pallas import tpu as pltpu


def kernel(x, dy):
    m, d = x.shape
    _, f = dy.shape
    half = d // 2

    def body(x_ref, dy_ref, out_ref, send_buf, recv_buf, send_sem, recv_sem):
        my_x = lax.axis_index("x")
        my_y = lax.axis_index("y")
        my_z = lax.axis_index("z")
        nbr = (1 - my_x, my_y, my_z)

        @pl.when(my_x == 0)
        def _():
            send_buf[...] = lax.dot_general(
                x_ref[:, half:], dy_ref[...],
                dimension_numbers=(((0,), (0,)), ((), ())),
                preferred_element_type=jnp.float32,
            )

        @pl.when(my_x == 1)
        def _():
            send_buf[...] = lax.dot_general(
                x_ref[:, :half], dy_ref[...],
                dimension_numbers=(((0,), (0,)), ((), ())),
                preferred_element_type=jnp.float32,
            )

        barrier_sem = pltpu.get_barrier_semaphore()
        pl.semaphore_signal(
            barrier_sem, inc=1, device_id=nbr,
            device_id_type=pl.DeviceIdType.MESH,
        )
        pl.semaphore_wait(barrier_sem, 1)

        rdma = pltpu.make_async_remote_copy(
            src_ref=send_buf,
            dst_ref=recv_buf,
            send_sem=send_sem,
            recv_sem=recv_sem,
            device_id=nbr,
            device_id_type=pl.DeviceIdType.MESH,
        )
        rdma.start()

        @pl.when(my_x == 0)
        def _():
            out_ref[...] = lax.dot_general(
                x_ref[:, :half], dy_ref[...],
                dimension_numbers=(((0,), (0,)), ((), ())),
                preferred_element_type=jnp.float32,
            )

        @pl.when(my_x == 1)
        def _():
            out_ref[...] = lax.dot_general(
                x_ref[:, half:], dy_ref[...],
                dimension_numbers=(((0,), (0,)), ((), ())),
                preferred_element_type=jnp.float32,
            )

        rdma.wait()
        out_ref[...] += recv_buf[...]

    return pl.pallas_call(
        body,
        out_shape=jax.ShapeDtypeStruct((half, f), jnp.float32),
        in_specs=[
            pl.BlockSpec(memory_space=pltpu.VMEM),
            pl.BlockSpec(memory_space=pltpu.VMEM),
        ],
        out_specs=pl.BlockSpec(memory_space=pltpu.VMEM),
        scratch_shapes=[
            pltpu.VMEM((half, f), jnp.float32),
            pltpu.VMEM((half, f), jnp.float32),
            pltpu.SemaphoreType.DMA,
            pltpu.SemaphoreType.DMA,
        ],
        compiler_params=pltpu.CompilerParams(collective_id=0),
    )(x, dy)


# baseline (device time: 115944 ns/iter reference)
import jax
import jax.numpy as jnp
from jax import lax
from jax.experimental import pallas as pl
from jax.experimental.pallas import tpu as pltpu


def kernel(x, dy):
    m, d = x.shape
    _, f = dy.shape
    half = d // 2

    def body(x_ref, dy_ref, out_ref, send_buf, recv_buf, send_sem, recv_sem):
        my_x = lax.axis_index("x")
        my_y = lax.axis_index("y")
        my_z = lax.axis_index("z")
        nbr = (1 - my_x, my_y, my_z)

        @pl.when(my_x == 0)
        def _():
            send_buf[...] = lax.dot_general(
                x_ref[:, half:], dy_ref[...],
                dimension_numbers=(((0,), (0,)), ((), ())),
                preferred_element_type=jnp.float32,
            )

        @pl.when(my_x == 1)
        def _():
            send_buf[...] = lax.dot_general(
                x_ref[:, :half], dy_ref[...],
                dimension_numbers=(((0,), (0,)), ((), ())),
                preferred_element_type=jnp.float32,
            )

        barrier_sem = pltpu.get_barrier_semaphore()
        pl.semaphore_signal(
            barrier_sem, inc=1, device_id=nbr,
            device_id_type=pl.DeviceIdType.MESH,
        )
        pl.semaphore_wait(barrier_sem, 1)

        rdma = pltpu.make_async_remote_copy(
            src_ref=send_buf,
            dst_ref=recv_buf,
            send_sem=send_sem,
            recv_sem=recv_sem,
            device_id=nbr,
            device_id_type=pl.DeviceIdType.MESH,
        )
        rdma.start()

        @pl.when(my_x == 0)
        def _():
            out_ref[...] = lax.dot_general(
                x_ref[:, :half], dy_ref[...],
                dimension_numbers=(((0,), (0,)), ((), ())),
                preferred_element_type=jnp.float32,
            )

        @pl.when(my_x == 1)
        def _():
            out_ref[...] = lax.dot_general(
                x_ref[:, half:], dy_ref[...],
                dimension_numbers=(((0,), (0,)), ((), ())),
                preferred_element_type=jnp.float32,
            )

        rdma.wait()
        out_ref[...] += recv_buf[...]

    return pl.pallas_call(
        body,
        out_shape=jax.ShapeDtypeStruct((half, f), jnp.float32),
        in_specs=[
            pl.BlockSpec(memory_space=pltpu.VMEM),
            pl.BlockSpec(memory_space=pltpu.VMEM),
        ],
        out_specs=pl.BlockSpec(memory_space=pltpu.VMEM),
        scratch_shapes=[
            pltpu.VMEM((half, f), jnp.float32),
            pltpu.VMEM((half, f), jnp.float32),
            pltpu.SemaphoreType.DMA,
            pltpu.SemaphoreType.DMA,
        ],
        compiler_params=pltpu.CompilerParams(
            collective_id=0, vmem_limit_bytes=100 * 1024 * 1024
        ),
    )(x, dy)


# device time: 64500 ns/iter; 1.7976x vs baseline; 1.7976x over previous
import jax
import jax.numpy as jnp
from jax import lax
from jax.experimental import pallas as pl
from jax.experimental.pallas import tpu as pltpu

S = 4

_DNUMS = (((0,), (0,)), ((), ()))
_MESH = pl.DeviceIdType.MESH


def kernel(x, dy):
    m, d = x.shape
    _, f = dy.shape
    half = d // 2
    fq = f // 4
    c = fq // S

    def body(x_ref, dy_ref, out_ref, part_ref, rxx_ref, red_ref,
             sx_send, sx_recv, sy1_send, sy1_recv, sz1_send, sz1_recv,
             s2_send, s2_recv):
        my_x = lax.axis_index("x")
        my_y = lax.axis_index("y")
        my_z = lax.axis_index("z")
        xn = (1 - my_x, my_y, my_z)
        yn = (my_x, 1 - my_y, my_z)
        zn = (my_x, my_y, 1 - my_z)
        q = 2 * my_y + my_z
        qy = 2 * (1 - my_y) + my_z
        qz = 2 * my_y + (1 - my_z)
        qd = 2 * (1 - my_y) + (1 - my_z)
        other_start = (1 - my_x) * half
        own_start = my_x * half

        barrier_sem = pltpu.get_barrier_semaphore()
        for nbr in (xn, yn, zn):
            pl.semaphore_signal(barrier_sem, inc=1, device_id=nbr,
                                device_id_type=_MESH)
        pl.semaphore_wait(barrier_sem, 3)

        def x_rdma(s):
            return pltpu.make_async_remote_copy(
                src_ref=part_ref.at[pl.ds(other_start, half), pl.ds(s * c, c)],
                dst_ref=rxx_ref.at[:, pl.ds(s * c, c)],
                send_sem=sx_send.at[s], recv_sem=sx_recv.at[s],
                device_id=xn, device_id_type=_MESH)

        def r1_rdma(s, nbr, send_sems, recv_sems):
            return pltpu.make_async_remote_copy(
                src_ref=red_ref.at[:, pl.ds(s * c, c)],
                dst_ref=out_ref.at[:, pl.ds(q * fq + s * c, c)],
                send_sem=send_sems.at[s], recv_sem=recv_sems.at[s],
                device_id=nbr, device_id_type=_MESH)

        def relay_rdma(s):
            slot = qz if s % 2 == 0 else qy
            nbr = yn if s % 2 == 0 else zn
            return pltpu.make_async_remote_copy(
                src_ref=out_ref.at[:, pl.ds(slot * fq + s * c, c)],
                dst_ref=out_ref.at[:, pl.ds(slot * fq + s * c, c)],
                send_sem=s2_send.at[s], recv_sem=s2_recv.at[s],
                device_id=nbr, device_id_type=_MESH)

        for s in range(S):
            for i in range(4):
                @pl.when(q == i)
                def _(s=s, i=i):
                    part_ref[:, s * c:(s + 1) * c] = lax.dot_general(
                        x_ref[...],
                        dy_ref[:, i * fq + s * c: i * fq + (s + 1) * c],
                        dimension_numbers=_DNUMS,
                        preferred_element_type=jnp.float32)
            x_rdma(s).start()

        for s in range(S):
            x_rdma(s).wait_recv()
            red_ref[:, s * c:(s + 1) * c] = (
                part_ref[pl.ds(own_start, half), s * c:(s + 1) * c]
                + rxx_ref[:, s * c:(s + 1) * c])
            for i in range(4):
                @pl.when(q == i)
                def _(s=s, i=i):
                    out_ref[:, i * fq + s * c: i * fq + (s + 1) * c] = (
                        red_ref[:, s * c:(s + 1) * c])
            r1_rdma(s, yn, sy1_send, sy1_recv).start()
            r1_rdma(s, zn, sz1_send, sz1_recv).start()

        for s in range(S):
            if s % 2 == 0:
                pltpu.make_async_remote_copy(
                    src_ref=red_ref.at[:, pl.ds(s * c, c)],
                    dst_ref=out_ref.at[:, pl.ds(qz * fq + s * c, c)],
                    send_sem=sz1_send.at[s], recv_sem=sz1_recv.at[s],
                    device_id=zn, device_id_type=_MESH).wait_recv()
            else:
                pltpu.make_async_remote_copy(
                    src_ref=red_ref.at[:, pl.ds(s * c, c)],
                    dst_ref=out_ref.at[:, pl.ds(qy * fq + s * c, c)],
                    send_sem=sy1_send.at[s], recv_sem=sy1_recv.at[s],
                    device_id=yn, device_id_type=_MESH).wait_recv()
            relay_rdma(s).start()

        for s in range(S):
            if s % 2 == 0:
                pltpu.make_async_remote_copy(
                    src_ref=red_ref.at[:, pl.ds(s * c, c)],
                    dst_ref=out_ref.at[:, pl.ds(qy * fq + s * c, c)],
                    send_sem=sy1_send.at[s], recv_sem=sy1_recv.at[s],
                    device_id=yn, device_id_type=_MESH).wait_recv()
            else:
                pltpu.make_async_remote_copy(
                    src_ref=red_ref.at[:, pl.ds(s * c, c)],
                    dst_ref=out_ref.at[:, pl.ds(qz * fq + s * c, c)],
                    send_sem=sz1_send.at[s], recv_sem=sz1_recv.at[s],
                    device_id=zn, device_id_type=_MESH).wait_recv()
            pltpu.make_async_remote_copy(
                src_ref=out_ref.at[:, pl.ds(qd * fq + s * c, c)],
                dst_ref=out_ref.at[:, pl.ds(qd * fq + s * c, c)],
                send_sem=s2_send.at[s], recv_sem=s2_recv.at[s],
                device_id=yn if s % 2 == 0 else zn,
                device_id_type=_MESH).wait_recv()
            x_rdma(s).wait_send()
            r1_rdma(s, yn, sy1_send, sy1_recv).wait_send()
            r1_rdma(s, zn, sz1_send, sz1_recv).wait_send()
            relay_rdma(s).wait_send()

    return pl.pallas_call(
        body,
        out_shape=jax.ShapeDtypeStruct((half, f), jnp.float32),
        in_specs=[
            pl.BlockSpec(memory_space=pltpu.VMEM),
            pl.BlockSpec(memory_space=pltpu.VMEM),
        ],
        out_specs=pl.BlockSpec(memory_space=pltpu.VMEM),
        scratch_shapes=[
            pltpu.VMEM((d, fq), jnp.float32),
            pltpu.VMEM((half, fq), jnp.float32),
            pltpu.VMEM((half, fq), jnp.float32),
            pltpu.SemaphoreType.DMA((S,)),
            pltpu.SemaphoreType.DMA((S,)),
            pltpu.SemaphoreType.DMA((S,)),
            pltpu.SemaphoreType.DMA((S,)),
            pltpu.SemaphoreType.DMA((S,)),
            pltpu.SemaphoreType.DMA((S,)),
            pltpu.SemaphoreType.DMA((S,)),
            pltpu.SemaphoreType.DMA((S,)),
        ],
        compiler_params=pltpu.CompilerParams(
            collective_id=0, vmem_limit_bytes=100 * 1024 * 1024
        ),
    )(x, dy)
